# baseline (speedup 1.0000x reference)
import functools

import jax
import jax.numpy as jnp
from jax import lax
from jax.experimental import pallas as pl
from jax.experimental.pallas import tpu as pltpu
from jax.experimental.pallas import tpu_sc as plsc

K = 100000
D = 16
B = 16384

NC = 2
NS = 16
NW = NC * NS
CH = B // NW
GCH = 128
NG = CH // GCH


def _sc_logreg(x_hbm, a_hbm, m_hbm, out_hbm, idx_v, rows_v, x_v, out_v, sem):
    cid = lax.axis_index("c")
    sid = lax.axis_index("s")
    wid = sid * NC + cid
    base = wid * CH

    pltpu.sync_copy(a_hbm.at[pl.ds(wid * NG, NG)], idx_v)

    copies = []
    for j in range(NG):
        copies.append(
            pltpu.async_copy(
                m_hbm.at[idx_v.at[j]], rows_v.at[pl.ds(j * GCH, GCH)], sem
            )
        )
    pltpu.sync_copy(x_hbm.at[pl.ds(base, CH)], x_v)
    for cp in copies:
        cp.wait()

    lanes = lax.iota(jnp.int32, 16)
    last_lane = lanes == 15

    def row(r, _):
        p = x_v[r] * rows_v[r]
        s = plsc.cumsum(p)
        plsc.store_scatter(out_v, [jnp.full((16,), r, jnp.int32)], s,
                           mask=last_lane)
        return _

    lax.fori_loop(0, CH, row, 0)

    def sig(blk, _):
        z = out_v[pl.ds(blk * 16, 16)]
        out_v[pl.ds(blk * 16, 16)] = 1.0 / (1.0 + jnp.exp(-z))
        return _

    lax.fori_loop(0, CH // 16, sig, 0)

    pltpu.sync_copy(out_v, out_hbm.at[pl.ds(base, CH)])


@functools.partial(
    pl.kernel,
    mesh=plsc.VectorSubcoreMesh(core_axis_name="c", subcore_axis_name="s"),
    compiler_params=pltpu.CompilerParams(
        needs_layout_passes=False, use_tc_tiling_on_sc=False
    ),
    out_type=jax.ShapeDtypeStruct((B,), jnp.float32),
    scratch_types=[
        pltpu.VMEM((NG, GCH), jnp.int32),
        pltpu.VMEM((CH, D), jnp.float32),
        pltpu.VMEM((CH, D), jnp.float32),
        pltpu.VMEM((CH,), jnp.float32),
        pltpu.SemaphoreType.DMA,
    ],
)
def _logreg_kernel(x_hbm, a_hbm, m_hbm, out_hbm, idx_v, rows_v, x_v, out_v, sem):
    _sc_logreg(x_hbm, a_hbm, m_hbm, out_hbm, idx_v, rows_v, x_v, out_v, sem)


def kernel(X, A, m):
    a2 = A.astype(jnp.int32).reshape(NW * NG, GCH)
    return _logreg_kernel(X, a2, m)


# speedup vs baseline: 1.0369x; 1.0369x over previous
import functools

import jax
import jax.numpy as jnp
from jax import lax
from jax.experimental import pallas as pl
from jax.experimental.pallas import tpu as pltpu
from jax.experimental.pallas import tpu_sc as plsc

K = 100000
D = 16
B = 16384

NC = 2
NS = 16
NW = NC * NS
CH = B // NW
GCH = 128
NG = CH // GCH


def _sc_logreg(x_hbm, a_hbm, m_hbm, out_hbm, idx_v, rows_v, x_v, out_v, sem):
    cid = lax.axis_index("c")
    sid = lax.axis_index("s")
    wid = sid * NC + cid
    base = wid * CH

    pltpu.sync_copy(a_hbm.at[pl.ds(wid * NG, NG)], idx_v)

    copies = []
    for j in range(NG):
        copies.append(
            pltpu.async_copy(
                m_hbm.at[idx_v.at[j]], rows_v.at[pl.ds(j * GCH, GCH)], sem
            )
        )
    pltpu.sync_copy(x_hbm.at[pl.ds(base, CH)], x_v)
    for cp in copies:
        cp.wait()

    lanes = lax.iota(jnp.int32, 16)

    def block(blk, _):
        row_ids = blk * 16 + lanes
        acc = jnp.zeros((16,), jnp.float32)
        for d in range(D):
            dcol = jnp.full((16,), d, jnp.int32)
            xv = plsc.load_gather(x_v, [row_ids, dcol])
            gv = plsc.load_gather(rows_v, [row_ids, dcol])
            acc = acc + xv * gv
        out_v[pl.ds(blk * 16, 16)] = 1.0 / (1.0 + jnp.exp(-acc))
        return _

    lax.fori_loop(0, CH // 16, block, 0)

    pltpu.sync_copy(out_v, out_hbm.at[pl.ds(base, CH)])


@functools.partial(
    pl.kernel,
    mesh=plsc.VectorSubcoreMesh(core_axis_name="c", subcore_axis_name="s"),
    compiler_params=pltpu.CompilerParams(
        needs_layout_passes=False, use_tc_tiling_on_sc=False
    ),
    out_type=jax.ShapeDtypeStruct((B,), jnp.float32),
    scratch_types=[
        pltpu.VMEM((NG, GCH), jnp.int32),
        pltpu.VMEM((CH, D), jnp.float32),
        pltpu.VMEM((CH, D), jnp.float32),
        pltpu.VMEM((CH,), jnp.float32),
        pltpu.SemaphoreType.DMA,
    ],
)
def _logreg_kernel(x_hbm, a_hbm, m_hbm, out_hbm, idx_v, rows_v, x_v, out_v, sem):
    _sc_logreg(x_hbm, a_hbm, m_hbm, out_hbm, idx_v, rows_v, x_v, out_v, sem)


def kernel(X, A, m):
    a2 = A.astype(jnp.int32).reshape(NW * NG, GCH)
    return _logreg_kernel(X, a2, m)


# speedup vs baseline: 1.0391x; 1.0021x over previous
import functools

import jax
import jax.numpy as jnp
from jax import lax
from jax.experimental import pallas as pl
from jax.experimental.pallas import tpu as pltpu
from jax.experimental.pallas import tpu_sc as plsc

K = 100000
D = 16
B = 16384

NC = 2
NS = 16
NW = NC * NS
CH = B // NW
GCH = 128
NG = CH // GCH


def _sc_logreg(x_hbm, a_hbm, m_hbm, out_hbm, idx_v, rows_v, x_v, out_v, sem):
    cid = lax.axis_index("c")
    sid = lax.axis_index("s")
    wid = sid * NC + cid
    base = wid * CH

    pltpu.sync_copy(a_hbm.at[pl.ds(base, CH)], idx_v)

    copies = []
    for j in range(NG):
        copies.append(
            pltpu.async_copy(
                m_hbm.at[idx_v.at[pl.ds(j * GCH, GCH)]],
                rows_v.at[pl.ds(j * GCH, GCH)],
                sem,
            )
        )
    pltpu.sync_copy(x_hbm.at[pl.ds(base, CH)], x_v)
    for cp in copies:
        cp.wait()

    lanes = lax.iota(jnp.int32, 16)

    def block(blk, _):
        row_ids = blk * 16 + lanes
        acc = jnp.zeros((16,), jnp.float32)
        for d in range(D):
            dcol = jnp.full((16,), d, jnp.int32)
            xv = plsc.load_gather(x_v, [row_ids, dcol])
            gv = plsc.load_gather(rows_v, [row_ids, dcol])
            acc = acc + xv * gv
        out_v[pl.ds(blk * 16, 16)] = 1.0 / (1.0 + jnp.exp(-acc))
        return _

    lax.fori_loop(0, CH // 16, block, 0)

    pltpu.sync_copy(out_v, out_hbm.at[pl.ds(base, CH)])


@functools.partial(
    pl.kernel,
    mesh=plsc.VectorSubcoreMesh(core_axis_name="c", subcore_axis_name="s"),
    compiler_params=pltpu.CompilerParams(
        needs_layout_passes=False, use_tc_tiling_on_sc=False
    ),
    out_type=jax.ShapeDtypeStruct((B,), jnp.float32),
    scratch_types=[
        pltpu.VMEM((CH,), jnp.int32),
        pltpu.VMEM((CH, D), jnp.float32),
        pltpu.VMEM((CH, D), jnp.float32),
        pltpu.VMEM((CH,), jnp.float32),
        pltpu.SemaphoreType.DMA,
    ],
)
def _logreg_kernel(x_hbm, a_hbm, m_hbm, out_hbm, idx_v, rows_v, x_v, out_v, sem):
    _sc_logreg(x_hbm, a_hbm, m_hbm, out_hbm, idx_v, rows_v, x_v, out_v, sem)


def kernel(X, A, m):
    return _logreg_kernel(X, A.astype(jnp.int32), m)
